# baseline (speedup 1.0000x reference)
import functools

import jax
import jax.numpy as jnp
from jax import lax
from jax.experimental import pallas as pl
from jax.experimental.pallas import tpu as pltpu
from jax.experimental.pallas import tpu_sc as plsc

_ALT_VALS = (150, 200, 250, 300)
_EMBED_D = 32
_BATCH = 16384
_LANES = 16
_NROWS = len(_ALT_VALS) + 1

_INFO = plsc.get_sparse_core_info()
_NC = _INFO.num_cores
_NS = _INFO.num_subcores
_NW = _NC * _NS
_BPW = _BATCH // _NW
_PPW = _BPW // 2
_PCHUNK = 64
_NCHUNK = _PPW // _PCHUNK
_PW = 2 * _EMBED_D


def _sc_body(alt_hbm, table_hbm, out_hbm, alt_v, idx_v, idxp_v, tab_v,
             pair_v, pairs_sh, rows_v, gsems, osem):
    cid = lax.axis_index("c")
    sid = lax.axis_index("s")
    wid = sid * _NC + cid
    base = wid * _BPW

    @pl.when(sid == 0)
    def _():
        pltpu.sync_copy(table_hbm, tab_v)
        for q in range(_NROWS * _NROWS):
            a, b = divmod(q, _NROWS)
            for h in range(_EMBED_D // _LANES):
                pair_v[q, pl.ds(h * _LANES, _LANES)] = (
                    tab_v[pl.ds(a * _EMBED_D + h * _LANES, _LANES)])
                pair_v[q, pl.ds(_EMBED_D + h * _LANES, _LANES)] = (
                    tab_v[pl.ds(b * _EMBED_D + h * _LANES, _LANES)])
        pltpu.sync_copy(pair_v, pairs_sh)

    pltpu.sync_copy(alt_hbm.at[pl.ds(base, _BPW)], alt_v)
    plsc.subcore_barrier()

    lane2 = lax.iota(jnp.int32, _LANES) * 2

    def compute_and_fire_gather(j):
        for i in range(2 * _PCHUNK // _LANES):
            off = j * 2 * _PCHUNK + i * _LANES
            a = alt_v[pl.ds(off, _LANES)]
            idx = jnp.where(a == _ALT_VALS[1], jnp.int32(1), jnp.int32(0))
            idx = jnp.where(a == _ALT_VALS[2], jnp.int32(2), idx)
            idx = jnp.where(a == _ALT_VALS[3], jnp.int32(3), idx)
            idx_v[pl.ds(off, _LANES)] = idx
        for m in range(_PCHUNK // _LANES):
            off = j * 2 * _PCHUNK + m * 2 * _LANES
            pa = plsc.load_gather(idx_v, [lane2 + off])
            pb = plsc.load_gather(idx_v, [lane2 + (off + 1)])
            idxp_v[pl.ds(j * _PCHUNK + m * _LANES, _LANES)] = (
                pa * _NROWS + pb)
        return pltpu.async_copy(
            pairs_sh.at[idxp_v.at[pl.ds(j * _PCHUNK, _PCHUNK)]],
            rows_v.at[pl.ds(j * _PCHUNK, _PCHUNK)],
            gsems[j],
        )

    def fire_writeback(j):
        return pltpu.async_copy(
            rows_v.at[pl.ds(j * _PCHUNK, _PCHUNK)],
            out_hbm.at[pl.ds(wid * _PPW + j * _PCHUNK, _PCHUNK)],
            osem,
        )

    gathers = [compute_and_fire_gather(0), compute_and_fire_gather(1)]
    writebacks = []
    for j in range(_NCHUNK):
        gathers[j].wait()
        writebacks.append(fire_writeback(j))
        if j + 2 < _NCHUNK:
            gathers.append(compute_and_fire_gather(j + 2))
    for w in writebacks:
        w.wait()


_sc_lookup = functools.partial(
    pl.kernel,
    mesh=plsc.VectorSubcoreMesh(core_axis_name="c", subcore_axis_name="s"),
    compiler_params=pltpu.CompilerParams(
        needs_layout_passes=False,
        skip_device_barrier=True,
        disable_bounds_checks=True,
        disable_semaphore_checks=True,
    ),
    out_type=jax.ShapeDtypeStruct((_BATCH // 2, _PW), jnp.float32),
    scratch_types=[
        pltpu.VMEM((_BPW,), jnp.int32),
        pltpu.VMEM((_BPW,), jnp.int32),
        pltpu.VMEM((_PPW,), jnp.int32),
        pltpu.VMEM((_NROWS * _EMBED_D,), jnp.float32),
        pltpu.VMEM((_NROWS * _NROWS, _PW), jnp.float32),
        pltpu.VMEM_SHARED((_NROWS * _NROWS, _PW), jnp.float32),
        pltpu.VMEM((_PPW, _PW), jnp.float32),
        [pltpu.SemaphoreType.DMA] * _NCHUNK,
        pltpu.SemaphoreType.DMA,
    ],
)(_sc_body)


def kernel(altitudes, embeddings):
    out = _sc_lookup(altitudes, embeddings.reshape(-1))
    return out.reshape(_BATCH, _EMBED_D)


# speedup vs baseline: 1.2370x; 1.2370x over previous
import functools

import jax
import jax.numpy as jnp
from jax import lax
from jax.experimental import pallas as pl
from jax.experimental.pallas import tpu as pltpu
from jax.experimental.pallas import tpu_sc as plsc

_ALT_VALS = (150, 200, 250, 300)
_EMBED_D = 32
_BATCH = 16384
_LANES = 16
_NROWS = len(_ALT_VALS) + 1

_INFO = plsc.get_sparse_core_info()
_NC = _INFO.num_cores
_NS = _INFO.num_subcores
_NW = _NC * _NS
_BPW = _BATCH // _NW
_CHUNK = 128
_NCHUNK = _BPW // _CHUNK


def _sc_body(alt_hbm, table_hbm, out_hbm, alt_v, idx_v, tab_l, table_v,
             rows_v, gsems, osem):
    sid = lax.axis_index("s")
    wid = sid * _NC + lax.axis_index("c")
    base = wid * _BPW

    @pl.when(lax.axis_index("s") == 0)
    def _():
        pltpu.sync_copy(table_hbm, tab_l)
        for c in range(_NS):
            pltpu.sync_copy(tab_l, table_v.at[pl.ds(c * _NROWS, _NROWS)])

    pltpu.sync_copy(alt_hbm.at[pl.ds(base, _BPW)], alt_v)
    plsc.subcore_barrier()

    def compute_and_fire_gather(j):
        for i in range(_CHUNK // _LANES):
            off = j * _CHUNK + i * _LANES
            a = alt_v[pl.ds(off, _LANES)]
            idx = jnp.where(a == _ALT_VALS[1], jnp.int32(1), jnp.int32(0))
            idx = jnp.where(a == _ALT_VALS[2], jnp.int32(2), idx)
            idx = jnp.where(a == _ALT_VALS[3], jnp.int32(3), idx)
            idx_v[pl.ds(off, _LANES)] = idx + sid * _NROWS
        return pltpu.async_copy(
            table_v.at[idx_v.at[pl.ds(j * _CHUNK, _CHUNK)]],
            rows_v.at[pl.ds(j * _CHUNK, _CHUNK)],
            gsems[j],
        )

    def fire_writeback(j):
        return pltpu.async_copy(
            rows_v.at[pl.ds(j * _CHUNK, _CHUNK)],
            out_hbm.at[pl.ds(base + j * _CHUNK, _CHUNK)],
            osem,
        )

    gathers = [compute_and_fire_gather(0), compute_and_fire_gather(1)]
    writebacks = []
    for j in range(_NCHUNK):
        gathers[j].wait()
        writebacks.append(fire_writeback(j))
        if j + 2 < _NCHUNK:
            gathers.append(compute_and_fire_gather(j + 2))
    for w in writebacks:
        w.wait()


_sc_lookup = functools.partial(
    pl.kernel,
    mesh=plsc.VectorSubcoreMesh(core_axis_name="c", subcore_axis_name="s"),
    compiler_params=pltpu.CompilerParams(
        skip_device_barrier=True,
        disable_bounds_checks=True,
        disable_semaphore_checks=True,
    ),
    out_type=jax.ShapeDtypeStruct((_BATCH, _EMBED_D), jnp.float32),
    scratch_types=[
        pltpu.VMEM((_BPW,), jnp.int32),
        pltpu.VMEM((_BPW,), jnp.int32),
        pltpu.VMEM((_NROWS, _EMBED_D), jnp.float32),
        pltpu.VMEM_SHARED((_NS * _NROWS, _EMBED_D), jnp.float32),
        pltpu.VMEM((_BPW, _EMBED_D), jnp.float32),
        [pltpu.SemaphoreType.DMA] * _NCHUNK,
        pltpu.SemaphoreType.DMA,
    ],
)(_sc_body)


def kernel(altitudes, embeddings):
    return _sc_lookup(altitudes, embeddings)


# speedup vs baseline: 1.2458x; 1.0072x over previous
import functools

import jax
import jax.numpy as jnp
from jax import lax
from jax.experimental import pallas as pl
from jax.experimental.pallas import tpu as pltpu
from jax.experimental.pallas import tpu_sc as plsc

_ALT_VALS = (150, 200, 250, 300)
_EMBED_D = 32
_BATCH = 16384
_LANES = 16
_NROWS = len(_ALT_VALS) + 1

_INFO = plsc.get_sparse_core_info()
_NC = _INFO.num_cores
_NS = _INFO.num_subcores
_NW = _NC * _NS
_BPW = _BATCH // _NW
_CHUNK = 128
_NCHUNK = _BPW // _CHUNK


def _sc_body(alt_hbm, table_hbm, out_hbm, alt_v, idx_v, table_v,
             rows_v, gsems, osem):
    sid = lax.axis_index("s")
    wid = sid * _NC + lax.axis_index("c")
    base = wid * _BPW

    pltpu.sync_copy(table_hbm, table_v.at[pl.ds(sid * _NROWS, _NROWS)])
    pltpu.sync_copy(alt_hbm.at[pl.ds(base, _BPW)], alt_v)

    def compute_and_fire_gather(j):
        for i in range(_CHUNK // _LANES):
            off = j * _CHUNK + i * _LANES
            a = alt_v[pl.ds(off, _LANES)]
            idx = jnp.where(a == _ALT_VALS[1], jnp.int32(1), jnp.int32(0))
            idx = jnp.where(a == _ALT_VALS[2], jnp.int32(2), idx)
            idx = jnp.where(a == _ALT_VALS[3], jnp.int32(3), idx)
            idx_v[pl.ds(off, _LANES)] = idx + sid * _NROWS
        return pltpu.async_copy(
            table_v.at[idx_v.at[pl.ds(j * _CHUNK, _CHUNK)]],
            rows_v.at[pl.ds(j * _CHUNK, _CHUNK)],
            gsems[j],
        )

    def fire_writeback(j):
        return pltpu.async_copy(
            rows_v.at[pl.ds(j * _CHUNK, _CHUNK)],
            out_hbm.at[pl.ds(base + j * _CHUNK, _CHUNK)],
            osem,
        )

    gathers = [compute_and_fire_gather(0), compute_and_fire_gather(1)]
    writebacks = []
    for j in range(_NCHUNK):
        gathers[j].wait()
        writebacks.append(fire_writeback(j))
        if j + 2 < _NCHUNK:
            gathers.append(compute_and_fire_gather(j + 2))
    for w in writebacks:
        w.wait()


_sc_lookup = functools.partial(
    pl.kernel,
    mesh=plsc.VectorSubcoreMesh(core_axis_name="c", subcore_axis_name="s"),
    compiler_params=pltpu.CompilerParams(
        skip_device_barrier=True,
        disable_bounds_checks=True,
        disable_semaphore_checks=True,
    ),
    out_type=jax.ShapeDtypeStruct((_BATCH, _EMBED_D), jnp.float32),
    scratch_types=[
        pltpu.VMEM((_BPW,), jnp.int32),
        pltpu.VMEM((_BPW,), jnp.int32),
        pltpu.VMEM_SHARED((_NS * _NROWS, _EMBED_D), jnp.float32),
        pltpu.VMEM((_BPW, _EMBED_D), jnp.float32),
        [pltpu.SemaphoreType.DMA] * _NCHUNK,
        pltpu.SemaphoreType.DMA,
    ],
)(_sc_body)


def kernel(altitudes, embeddings):
    return _sc_lookup(altitudes, embeddings)
